# baseline (speedup 1.0000x reference)
import functools
import jax
import jax.numpy as jnp
from jax import lax
from jax.experimental import pallas as pl
from jax.experimental.pallas import tpu as pltpu
from jax.experimental.pallas import tpu_sc as plsc

B = 2
N = 200000
NX = 512
NY = 512
CELLS = NX * NY
MAXV = 20000
MAXP = 30
NSUB = 16
PER_SUB = CELLS // NSUB
VOX_ROWS = B * MAXV * MAXP
DUMP = VOX_ROWS
CH = 320
NCHUNK = N // CH
GRP = 64
ZROWS = 1500

XMIN = -51.2
YMIN = -51.2
ZMIN = -5.0
VSX = 0.2
VSY = 0.2
VSZ = 8.0


def _build_kernel():
    mesh = plsc.VectorSubcoreMesh(core_axis_name="c", subcore_axis_name="s")

    @functools.partial(
        pl.kernel,
        out_type=[
            jax.ShapeDtypeStruct((VOX_ROWS * 4 + 32,), jnp.float32),
            jax.ShapeDtypeStruct((B, MAXV, 4), jnp.int32),
            jax.ShapeDtypeStruct((B, MAXV, 1), jnp.int32),
        ],
        mesh=mesh,
        compiler_params=pltpu.CompilerParams(
            needs_layout_passes=False, use_tc_tiling_on_sc=False
        ),
        scratch_types=[
            pltpu.VMEM((PER_SUB,), jnp.int32),
            pltpu.VMEM((PER_SUB,), jnp.int32),
            pltpu.VMEM((PER_SUB + 128,), jnp.int32),
            pltpu.VMEM((CH * 4,), jnp.float32),
            pltpu.VMEM((128,), jnp.int32),
            pltpu.VMEM((128,), jnp.float32),
            pltpu.VMEM((ZROWS * 4,), jnp.float32),
            pltpu.VMEM((128, 4), jnp.int32),
            pltpu.VMEM((128, 1), jnp.int32),
            pltpu.VMEM((16,), jnp.int32),
            pltpu.VMEM((16, 16), jnp.int32),
            pltpu.VMEM_SHARED((16, 16), jnp.int32),
        ],
    )
    def vox_kernel(pts_hbm, vox_hbm, coors_hbm, npts_hbm,
                   tblA, tblB, uniq, pts, idxb, valsb, zbuf, cbuf, nbuf,
                   tmp16, tot2d, sh_tot):
        cid = lax.axis_index("c")
        sid = lax.axis_index("s")
        bb = cid
        lane = lax.iota(jnp.int32, 16)
        lo = sid * PER_SUB
        vox_base = bb * (MAXV * MAXP)
        zeros16 = jnp.zeros((16,), jnp.int32)
        ones16 = jnp.ones((16,), jnp.int32)

        def cells_of(v):
            pbase = v * 64 + lane * 4
            fx = plsc.load_gather(pts, [pbase])
            fy = plsc.load_gather(pts, [pbase + 1])
            fz = plsc.load_gather(pts, [pbase + 2])
            gx = (fx - XMIN) / VSX
            gy = (fy - YMIN) / VSY
            gz = (fz - ZMIN) / VSZ
            valid = ((gx >= 0.0) & (gx < float(NX)) &
                     (gy >= 0.0) & (gy < float(NY)) &
                     (gz >= 0.0) & (gz < 1.0))
            cx = gx.astype(jnp.int32)
            cy = gy.astype(jnp.int32)
            cell = cy * NX + cx
            return cell, valid

        def _z(i, _):
            tblA[pl.ds(i * 16, 16)] = zeros16
            tblB[pl.ds(i * 16, 16)] = zeros16
            return 0
        lax.fori_loop(0, PER_SUB // 16, _z, 0)

        zf = jnp.zeros((16,), jnp.float32)

        def _zb(i, _):
            zbuf[pl.ds(i * 16, 16)] = zf
            return 0
        lax.fori_loop(0, ZROWS * 4 // 16, _zb, 0)
        zoff = vox_base * 4 + sid * (ZROWS * 4 * 25)
        for j in range(25):
            pltpu.sync_copy(zbuf, vox_hbm.at[pl.ds(zoff + j * ZROWS * 4, ZROWS * 4)])

        bvec = jnp.full((16,), bb, jnp.int32)
        mones = jnp.full((16,), -1, jnp.int32)
        for g in range(8):
            rows = g * 16 + lane
            plsc.store_scatter(cbuf, [rows, zeros16], bvec)
            for col in (1, 2, 3):
                plsc.store_scatter(cbuf, [rows, jnp.full((16,), col, jnp.int32)], mones)
            plsc.store_scatter(nbuf, [rows, zeros16], zeros16)
        for j in range(10):
            roff = sid * 1250 + j * 125
            pltpu.sync_copy(cbuf.at[pl.ds(0, 125)], coors_hbm.at[bb].at[pl.ds(roff, 125)])
            pltpu.sync_copy(nbuf.at[pl.ds(0, 125)], npts_hbm.at[bb].at[pl.ds(roff, 125)])

        def _pa(ci, _):
            pltpu.sync_copy(pts_hbm.at[bb].at[pl.ds(ci * CH * 4, CH * 4)], pts)
            for v in range(CH // 16):
                cell, valid = cells_of(v)
                m = valid & (cell >= lo) & (cell < lo + PER_SUB)
                loc = jnp.clip(cell - lo, 0, PER_SUB - 1)
                plsc.store_scatter(tblA, [loc], ones16, mask=m)
            return 0
        lax.fori_loop(0, NCHUNK, _pa, 0)

        def _b1(i, acc):
            return acc + tblA[pl.ds(i * 16, 16)]
        acc = lax.fori_loop(0, PER_SUB // 16, _b1, zeros16)
        total = jnp.sum(acc)
        tmp16[...] = jnp.full((16,), total, jnp.int32)
        pltpu.sync_copy(tmp16, sh_tot.at[sid])
        plsc.subcore_barrier()
        pltpu.sync_copy(sh_tot, tot2d)
        tv = plsc.load_gather(tot2d, [lane, lane])
        base = jnp.sum(jnp.where(lane < sid, tv, 0))

        def _b2(i, cnt_so_far):
            occ = tblA[pl.ds(i * 16, 16)]
            inc = plsc.cumsum(occ)
            excl = inc - occ
            tblB[pl.ds(i * 16, 16)] = base + cnt_so_far + excl
            occm = occ > 0
            plsc.store_compressed(uniq.at[pl.ds(cnt_so_far, 16)], i * 16 + lane,
                                  mask=occm)
            return cnt_so_far + jnp.sum(occ)
        cnt_s = lax.fori_loop(0, PER_SUB // 16, _b2, 0)

        def _za(i, _):
            tblA[pl.ds(i * 16, 16)] = zeros16
            return 0
        lax.fori_loop(0, PER_SUB // 16, _za, 0)

        plsc.subcore_barrier()

        def _pc(ci, _):
            pltpu.sync_copy(pts_hbm.at[bb].at[pl.ds(ci * CH * 4, CH * 4)], pts)
            for g in range(CH // 32):
                for q in range(2):
                    v = g * 2 + q
                    cell, valid = cells_of(v)
                    m = valid & (cell >= lo) & (cell < lo + PER_SUB)
                    loc = jnp.clip(cell - lo, 0, PER_SUB - 1)
                    cnt1, lastm = plsc.scan_count(cell, mask=m)
                    basev = plsc.load_gather(tblA, [loc], mask=m)
                    rank = basev + cnt1 - 1
                    plsc.store_scatter(tblA, [loc], basev + cnt1, mask=m & lastm)
                    slotv = plsc.load_gather(tblB, [loc], mask=m)
                    keep = m & (rank < MAXP) & (slotv < MAXV)
                    dest = jnp.where(keep, vox_base + slotv * MAXP + rank, DUMP)
                    for c in range(4):
                        plsc.store_scatter(
                            idxb, [q * 64 + lane * 4 + c], dest * 4 + c)
                for k in range(8):
                    valsb[pl.ds(k * 16, 16)] = pts[pl.ds(g * 128 + k * 16, 16)]
                pltpu.sync_copy(valsb, vox_hbm.at[idxb])
            return 0
        lax.fori_loop(0, NCHUNK, _pc, 0)

        L = jnp.clip(MAXV - base, 0, cnt_s)

        def _fill_stage(j):
            for g in range(8):
                posv = j * 128 + g * 16 + lane
                mm = posv < L
                lidx = plsc.load_gather(uniq, [jnp.clip(posv, 0, PER_SUB + 111)],
                                        mask=mm)
                cell = lo + lidx
                yv = cell >> 9
                xv = cell & (NX - 1)
                rows = g * 16 + lane
                plsc.store_scatter(cbuf, [rows, zeros16], bvec)
                plsc.store_scatter(cbuf, [rows, ones16], zeros16)
                plsc.store_scatter(cbuf, [rows, jnp.full((16,), 2, jnp.int32)], yv)
                plsc.store_scatter(cbuf, [rows, jnp.full((16,), 3, jnp.int32)], xv)
                cv = plsc.load_gather(tblA, [jnp.clip(lidx, 0, PER_SUB - 1)],
                                      mask=mm)
                plsc.store_scatter(nbuf, [rows, zeros16],
                                   jnp.minimum(cv, MAXP))
            return None

        nfull = L // 128

        def _wf(j, _):
            _fill_stage(j)
            roff = base + j * 128
            pltpu.sync_copy(cbuf, coors_hbm.at[bb].at[pl.ds(roff, 128)])
            pltpu.sync_copy(nbuf, npts_hbm.at[bb].at[pl.ds(roff, 128)])
            return 0
        lax.fori_loop(0, nfull, _wf, 0)

        rem = L - nfull * 128

        @pl.when(rem > 0)
        def _tail():
            _fill_stage(nfull)
            t = jnp.int32(0)
            for sz in (64, 32, 16, 8, 4, 2, 1):
                take = (rem & sz) > 0
                tt = t

                @pl.when(take)
                def _():
                    roff = base + nfull * 128 + tt
                    pltpu.sync_copy(cbuf.at[pl.ds(tt, sz)],
                                    coors_hbm.at[bb].at[pl.ds(roff, sz)])
                    pltpu.sync_copy(nbuf.at[pl.ds(tt, sz)],
                                    npts_hbm.at[bb].at[pl.ds(roff, sz)])
                t = t + jnp.where(take, sz, 0)

    return vox_kernel


@jax.jit
def kernel(points_lst):
    vox, coors, npts = _build_kernel()(points_lst.reshape(B, N * 4))
    voxel_feature = vox[:VOX_ROWS * 4].reshape(B * MAXV, MAXP, 4)
    coors_batch = coors.reshape(B * MAXV, 4).astype(jnp.int64)
    num_points_per_voxel = npts.reshape(B * MAXV)
    return voxel_feature, coors_batch, num_points_per_voxel
